# baseline (speedup 1.0000x reference)
import functools

import jax
import jax.numpy as jnp
from jax import lax
from jax.experimental import pallas as pl
from jax.experimental.pallas import tpu as pltpu
from jax.experimental.pallas import tpu_sc as plsc

_INFO = plsc.get_sparse_core_info()
_NC = _INFO.num_cores
_NS = _INFO.num_subcores
_NW = _NC * _NS

_CHUNK = 128


def _embed_kernel(total_rows, embed_dim, n_chunks):
    rows_per_w = n_chunks * _CHUNK
    mesh = plsc.VectorSubcoreMesh(core_axis_name="c", subcore_axis_name="s")

    @functools.partial(
        pl.kernel,
        mesh=mesh,
        out_type=jax.ShapeDtypeStruct((total_rows, embed_dim), jnp.float32),
        scratch_types=[
            pltpu.VMEM((n_chunks, _CHUNK), jnp.int32),
            pltpu.VMEM((_CHUNK, embed_dim), jnp.float32),
            pltpu.SemaphoreType.DMA,
        ],
        compiler_params=pltpu.CompilerParams(use_tc_tiling_on_sc=False),
    )
    def k(ids_hbm, table_hbm, out_hbm, idx_v, rows_v, sem):
        wid = lax.axis_index("s") * _NC + lax.axis_index("c")
        base = wid * rows_per_w
        pltpu.sync_copy(ids_hbm.at[wid], idx_v)

        def chunk(j, carry):
            pltpu.async_copy(table_hbm.at[idx_v.at[j]], rows_v, sem).wait()
            pltpu.sync_copy(rows_v, out_hbm.at[pl.ds(base + j * _CHUNK, _CHUNK)])
            return carry

        lax.fori_loop(0, n_chunks, chunk, 0, unroll=False)

    return k


def kernel(ids, weight):
    batch, seq = ids.shape
    vocab, dim = weight.shape
    total = batch * seq
    assert total % (_NW * _CHUNK) == 0
    n_chunks = total // (_NW * _CHUNK)
    ids_grouped = ids.reshape(_NW, n_chunks, _CHUNK).astype(jnp.int32)
    out = _embed_kernel(total, dim, n_chunks)(ids_grouped, weight)
    return out.reshape(batch, seq, dim)


# speedup vs baseline: 1.0464x; 1.0464x over previous
import functools

import jax
import jax.numpy as jnp
from jax import lax
from jax.experimental import pallas as pl
from jax.experimental.pallas import tpu as pltpu
from jax.experimental.pallas import tpu_sc as plsc

_INFO = plsc.get_sparse_core_info()
_NC = _INFO.num_cores
_NS = _INFO.num_subcores
_NW = _NC * _NS

_CHUNK = 128
_M = 10


def _embed_kernel(total_rows, embed_dim, n_chunks):
    rows_per_w = n_chunks * _CHUNK
    n_groups = n_chunks // _M
    mesh = plsc.VectorSubcoreMesh(core_axis_name="c", subcore_axis_name="s")

    scratch = (
        [pltpu.VMEM((n_chunks, _CHUNK), jnp.int32)]
        + [pltpu.VMEM((_CHUNK, embed_dim), jnp.float32) for _ in range(_M)]
        + [pltpu.SemaphoreType.DMA for _ in range(2 * _M)]
    )

    @functools.partial(
        pl.kernel,
        mesh=mesh,
        out_type=jax.ShapeDtypeStruct((total_rows, embed_dim), jnp.float32),
        scratch_types=scratch,
        compiler_params=pltpu.CompilerParams(use_tc_tiling_on_sc=False),
    )
    def k(ids_hbm, table_hbm, out_hbm, idx_v, *rest):
        bufs = rest[:_M]
        gs = rest[_M:2 * _M]
        ws = rest[2 * _M:]
        wid = lax.axis_index("s") * _NC + lax.axis_index("c")
        base = wid * rows_per_w
        pltpu.sync_copy(ids_hbm.at[wid], idx_v)

        for b in range(_M):
            pltpu.async_copy(table_hbm.at[idx_v.at[b]], bufs[b], gs[b])

        def group(g, carry):
            for b in range(_M):
                jj = g * _M + b
                pltpu.make_async_copy(table_hbm.at[idx_v.at[b]], bufs[b], gs[b]).wait()
                pltpu.async_copy(
                    bufs[b], out_hbm.at[pl.ds(base + jj * _CHUNK, _CHUNK)], ws[b])
            for b in range(_M):
                jj = (g + 1) * _M + b
                pltpu.make_async_copy(
                    bufs[b], out_hbm.at[pl.ds(base, _CHUNK)], ws[b]).wait()
                pltpu.async_copy(table_hbm.at[idx_v.at[jj]], bufs[b], gs[b])
            return carry

        lax.fori_loop(0, n_groups - 1, group, 0, unroll=False)

        g_last = n_groups - 1
        for b in range(_M):
            jj = g_last * _M + b
            pltpu.make_async_copy(table_hbm.at[idx_v.at[b]], bufs[b], gs[b]).wait()
            pltpu.async_copy(
                bufs[b], out_hbm.at[pl.ds(base + jj * _CHUNK, _CHUNK)], ws[b])
        for b in range(_M):
            pltpu.make_async_copy(
                bufs[b], out_hbm.at[pl.ds(base, _CHUNK)], ws[b]).wait()

    return k


def kernel(ids, weight):
    batch, seq = ids.shape
    vocab, dim = weight.shape
    total = batch * seq
    assert total % (_NW * _CHUNK * _M) == 0
    n_chunks = total // (_NW * _CHUNK)
    ids_grouped = ids.reshape(_NW, n_chunks, _CHUNK).astype(jnp.int32)
    out = _embed_kernel(total, dim, n_chunks)(ids_grouped, weight)
    return out.reshape(batch, seq, dim)


# speedup vs baseline: 1.1687x; 1.1168x over previous
import functools

import jax
import jax.numpy as jnp
from jax import lax
from jax.experimental import pallas as pl
from jax.experimental.pallas import tpu as pltpu
from jax.experimental.pallas import tpu_sc as plsc

_INFO = plsc.get_sparse_core_info()
_NC = _INFO.num_cores
_NS = _INFO.num_subcores
_NW = _NC * _NS

_CHUNK = 128
_M = 10

_TBLK = 2048


def _relayout_table(wt):
    embed, vocab = wt.shape

    def body(in_ref, out_ref):
        t = in_ref[...].T
        out_ref[...] = jnp.concatenate([t, t], axis=1)

    return pl.pallas_call(
        body,
        grid=(pl.cdiv(vocab, _TBLK),),
        in_specs=[pl.BlockSpec((embed, _TBLK), lambda i: (0, i))],
        out_specs=pl.BlockSpec((_TBLK, 2 * embed), lambda i: (i, 0)),
        out_shape=jax.ShapeDtypeStruct((vocab, 2 * embed), jnp.float32),
    )(wt)


def _gather_rows(total_rows, embed_dim, n_chunks):
    rows_per_w = n_chunks * _CHUNK
    n_groups = n_chunks // _M
    mesh = plsc.VectorSubcoreMesh(core_axis_name="c", subcore_axis_name="s")

    scratch = (
        [pltpu.VMEM((n_chunks, _CHUNK), jnp.int32)]
        + [pltpu.VMEM((_CHUNK, embed_dim), jnp.float32) for _ in range(_M)]
        + [pltpu.SemaphoreType.DMA for _ in range(2 * _M)]
    )

    @functools.partial(
        pl.kernel,
        mesh=mesh,
        out_type=jax.ShapeDtypeStruct((total_rows, embed_dim), jnp.float32),
        scratch_types=scratch,
        compiler_params=pltpu.CompilerParams(use_tc_tiling_on_sc=False),
    )
    def k(ids_hbm, table_hbm, out_hbm, idx_v, *rest):
        bufs = rest[:_M]
        gs = rest[_M:2 * _M]
        ws = rest[2 * _M:]
        wid = lax.axis_index("s") * _NC + lax.axis_index("c")
        base = wid * rows_per_w
        pltpu.sync_copy(ids_hbm.at[pl.ds(wid * n_chunks, n_chunks)], idx_v)

        for b in range(_M):
            pltpu.async_copy(table_hbm.at[idx_v.at[b]], bufs[b], gs[b])

        def group(g, carry):
            for b in range(_M):
                jj = g * _M + b
                pltpu.make_async_copy(table_hbm.at[idx_v.at[b]], bufs[b], gs[b]).wait()
                pltpu.async_copy(
                    bufs[b], out_hbm.at[pl.ds(base + jj * _CHUNK, _CHUNK)], ws[b])
            for b in range(_M):
                jj = (g + 1) * _M + b
                pltpu.make_async_copy(
                    bufs[b], out_hbm.at[pl.ds(base, _CHUNK)], ws[b]).wait()
                pltpu.async_copy(table_hbm.at[idx_v.at[jj]], bufs[b], gs[b])
            return carry

        lax.fori_loop(0, n_groups - 1, group, 0, unroll=False)

        g_last = n_groups - 1
        for b in range(_M):
            jj = g_last * _M + b
            pltpu.make_async_copy(table_hbm.at[idx_v.at[b]], bufs[b], gs[b]).wait()
            pltpu.async_copy(
                bufs[b], out_hbm.at[pl.ds(base + jj * _CHUNK, _CHUNK)], ws[b])
        for b in range(_M):
            pltpu.make_async_copy(
                bufs[b], out_hbm.at[pl.ds(base, _CHUNK)], ws[b]).wait()

    return k


def _format_output(rows2, seq, batch, embed):
    half = batch // 2

    def body(in_ref, out_ref):
        x = in_ref[...]
        a = x[:, :embed].T
        b = x[:, embed:].T
        out_ref[0] = jnp.concatenate([a, b], axis=1)

    return pl.pallas_call(
        body,
        grid=(seq,),
        in_specs=[pl.BlockSpec((half, 2 * embed), lambda s: (s, 0))],
        out_specs=pl.BlockSpec((1, embed, batch), lambda s: (s, 0, 0)),
        out_shape=jax.ShapeDtypeStruct((seq, embed, batch), jnp.float32),
    )(rows2)


def kernel(ids, weight):
    batch, seq = ids.shape
    vocab, embed = weight.shape
    total = batch * seq
    assert total % (_NW * _CHUNK * _M) == 0
    n_chunks = total // (_NW * _CHUNK)

    table2 = _relayout_table(weight.T).reshape(2 * vocab, embed)
    ids_perm = ids.T.reshape(seq, 2, batch // 2).transpose(0, 2, 1)
    ids_flat = (ids_perm.reshape(total) * 2).reshape(total // _CHUNK, _CHUNK)
    rows = _gather_rows(total, embed, n_chunks)(ids_flat, table2)
    rows2 = rows.reshape(total // 2, 2 * embed)
    o_phys = _format_output(rows2, seq, batch, embed)
    return jnp.transpose(o_phys, (2, 0, 1))


# speedup vs baseline: 2.0689x; 1.7703x over previous
import functools

import jax
import jax.numpy as jnp
from jax import lax
from jax.experimental import pallas as pl
from jax.experimental.pallas import tpu as pltpu
from jax.experimental.pallas import tpu_sc as plsc

_INFO = plsc.get_sparse_core_info()
_NC = _INFO.num_cores
_NS = _INFO.num_subcores
_NW = _NC * _NS

_CHUNK = 128
_M = 10

_PBLK = 2048


def _relayout_table(wt):
    embed, vocab = wt.shape
    n_pairs = pl.cdiv(vocab, 2 * _PBLK)

    def body(in1_ref, in2_ref, out_ref):
        out_ref[...] = jnp.concatenate(
            [in1_ref[...].T, in2_ref[...].T], axis=1)

    return pl.pallas_call(
        body,
        grid=(n_pairs,),
        in_specs=[
            pl.BlockSpec((embed, _PBLK), lambda i: (0, 2 * i)),
            pl.BlockSpec(
                (embed, _PBLK),
                lambda i: (0, jnp.minimum(2 * i + 1,
                                          pl.cdiv(vocab, _PBLK) - 1))),
        ],
        out_specs=pl.BlockSpec((_PBLK, 2 * embed), lambda i: (i, 0)),
        out_shape=jax.ShapeDtypeStruct((n_pairs * _PBLK, 2 * embed),
                                       jnp.float32),
    )(wt, wt)


def _gather_rows(total_rows, embed_dim, n_chunks):
    rows_per_w = n_chunks * _CHUNK
    n_groups = n_chunks // _M
    mesh = plsc.VectorSubcoreMesh(core_axis_name="c", subcore_axis_name="s")

    scratch = (
        [pltpu.VMEM((n_chunks, _CHUNK), jnp.int32)]
        + [pltpu.VMEM((_CHUNK, embed_dim), jnp.float32) for _ in range(_M)]
        + [pltpu.SemaphoreType.DMA for _ in range(2 * _M)]
    )

    @functools.partial(
        pl.kernel,
        mesh=mesh,
        out_type=jax.ShapeDtypeStruct((total_rows, embed_dim), jnp.float32),
        scratch_types=scratch,
        compiler_params=pltpu.CompilerParams(use_tc_tiling_on_sc=False),
    )
    def k(ids_hbm, table_hbm, out_hbm, idx_v, *rest):
        bufs = rest[:_M]
        gs = rest[_M:2 * _M]
        ws = rest[2 * _M:]
        wid = lax.axis_index("s") * _NC + lax.axis_index("c")
        base = wid * rows_per_w
        pltpu.sync_copy(ids_hbm.at[pl.ds(wid * n_chunks, n_chunks)], idx_v)

        for b in range(_M):
            pltpu.async_copy(table_hbm.at[idx_v.at[b]], bufs[b], gs[b])

        def group(g, carry):
            for b in range(_M):
                jj = g * _M + b
                pltpu.make_async_copy(table_hbm.at[idx_v.at[b]], bufs[b], gs[b]).wait()
                pltpu.async_copy(
                    bufs[b], out_hbm.at[pl.ds(base + jj * _CHUNK, _CHUNK)], ws[b])
            for b in range(_M):
                jj = (g + 1) * _M + b
                pltpu.make_async_copy(
                    bufs[b], out_hbm.at[pl.ds(base, _CHUNK)], ws[b]).wait()
                pltpu.async_copy(table_hbm.at[idx_v.at[jj]], bufs[b], gs[b])
            return carry

        lax.fori_loop(0, n_groups - 1, group, 0, unroll=False)

        g_last = n_groups - 1
        for b in range(_M):
            jj = g_last * _M + b
            pltpu.make_async_copy(table_hbm.at[idx_v.at[b]], bufs[b], gs[b]).wait()
            pltpu.async_copy(
                bufs[b], out_hbm.at[pl.ds(base + jj * _CHUNK, _CHUNK)], ws[b])
        for b in range(_M):
            pltpu.make_async_copy(
                bufs[b], out_hbm.at[pl.ds(base, _CHUNK)], ws[b]).wait()

    return k


def _format_output(rows2, seq, batch, embed):
    half = batch // 2

    def body(in_ref, out_ref):
        x = in_ref[...]
        a = x[:, :embed].T
        b = x[:, embed:].T
        out_ref[0] = jnp.concatenate([a, b], axis=1)

    return pl.pallas_call(
        body,
        grid=(seq,),
        in_specs=[pl.BlockSpec((half, 2 * embed), lambda s: (s, 0))],
        out_specs=pl.BlockSpec((1, embed, batch), lambda s: (s, 0, 0)),
        out_shape=jax.ShapeDtypeStruct((seq, embed, batch), jnp.float32),
    )(rows2)


def kernel(ids, weight):
    batch, seq = ids.shape
    vocab, embed = weight.shape
    total = batch * seq
    assert total % (_NW * _CHUNK * _M) == 0
    n_chunks = total // (_NW * _CHUNK)

    packed = _relayout_table(weight.T)
    table2 = packed.reshape(2 * packed.shape[0], embed)
    ids_perm = ids.T.reshape(seq, 2, batch // 2).transpose(0, 2, 1)
    idp = ids_perm.reshape(total)
    blk = idp // _PBLK
    idp = (blk // 2) * (2 * _PBLK) + (idp % _PBLK) * 2 + (blk % 2)
    ids_flat = idp.reshape(total // _CHUNK, _CHUNK)
    rows = _gather_rows(total, embed, n_chunks)(ids_flat, table2)
    rows2 = rows.reshape(total // 2, 2 * embed)
    o_phys = _format_output(rows2, seq, batch, embed)
    return jnp.transpose(o_phys, (2, 0, 1))


# speedup vs baseline: 2.1786x; 1.0530x over previous
import functools

import jax
import jax.numpy as jnp
from jax import lax
from jax.experimental import pallas as pl
from jax.experimental.pallas import tpu as pltpu
from jax.experimental.pallas import tpu_sc as plsc

_INFO = plsc.get_sparse_core_info()
_NC = _INFO.num_cores
_NS = _INFO.num_subcores
_NW = _NC * _NS

_CHUNK = 128
_M = 10

_PBLK = 2048


def _relayout_table(wt):
    embed, vocab = wt.shape
    n_pairs = pl.cdiv(vocab, 2 * _PBLK)

    def body(in1_ref, in2_ref, out_ref):
        out_ref[...] = jnp.concatenate(
            [in1_ref[...].T, in2_ref[...].T], axis=1)

    return pl.pallas_call(
        body,
        grid=(n_pairs,),
        in_specs=[
            pl.BlockSpec((embed, _PBLK), lambda i: (0, 2 * i)),
            pl.BlockSpec(
                (embed, _PBLK),
                lambda i: (0, jnp.minimum(2 * i + 1,
                                          pl.cdiv(vocab, _PBLK) - 1))),
        ],
        out_specs=pl.BlockSpec((_PBLK, 2 * embed), lambda i: (i, 0)),
        out_shape=jax.ShapeDtypeStruct((n_pairs * _PBLK, 2 * embed),
                                       jnp.float32),
    )(wt, wt)


def _gather_rows(total_rows, embed_dim, n_chunks):
    rows_per_w = n_chunks * _CHUNK
    n_groups = n_chunks // _M
    mesh = plsc.VectorSubcoreMesh(core_axis_name="c", subcore_axis_name="s")

    scratch = (
        [pltpu.VMEM((n_chunks, _CHUNK), jnp.int32)]
        + [pltpu.VMEM((_CHUNK, embed_dim), jnp.float32) for _ in range(_M)]
        + [pltpu.SemaphoreType.DMA for _ in range(2 * _M)]
    )

    @functools.partial(
        pl.kernel,
        mesh=mesh,
        out_type=jax.ShapeDtypeStruct((total_rows, embed_dim), jnp.float32),
        scratch_types=scratch,
        compiler_params=pltpu.CompilerParams(use_tc_tiling_on_sc=False),
    )
    def k(ids_hbm, table_hbm, out_hbm, idx_v, *rest):
        bufs = rest[:_M]
        gs = rest[_M:2 * _M]
        ws = rest[2 * _M:]
        wid = lax.axis_index("s") * _NC + lax.axis_index("c")
        base = wid * rows_per_w
        pltpu.sync_copy(ids_hbm.at[pl.ds(wid * n_chunks, n_chunks)], idx_v)

        for b in range(_M):
            pltpu.async_copy(table_hbm.at[idx_v.at[b]], bufs[b], gs[b])

        def group(g, carry):
            for b in range(_M):
                jj = g * _M + b
                pltpu.make_async_copy(table_hbm.at[idx_v.at[b]], bufs[b], gs[b]).wait()
                pltpu.async_copy(
                    bufs[b], out_hbm.at[pl.ds(base + jj * _CHUNK, _CHUNK)], ws[b])
            for b in range(_M):
                jj = (g + 1) * _M + b
                pltpu.make_async_copy(
                    bufs[b], out_hbm.at[pl.ds(base, _CHUNK)], ws[b]).wait()
                pltpu.async_copy(table_hbm.at[idx_v.at[jj]], bufs[b], gs[b])
            return carry

        lax.fori_loop(0, n_groups - 1, group, 0, unroll=False)

        g_last = n_groups - 1
        for b in range(_M):
            jj = g_last * _M + b
            pltpu.make_async_copy(table_hbm.at[idx_v.at[b]], bufs[b], gs[b]).wait()
            pltpu.async_copy(
                bufs[b], out_hbm.at[pl.ds(base + jj * _CHUNK, _CHUNK)], ws[b])
        for b in range(_M):
            pltpu.make_async_copy(
                bufs[b], out_hbm.at[pl.ds(base, _CHUNK)], ws[b]).wait()

    return k


def _format_output(rows2, seq, batch, embed, sper=5):
    half = batch // 2

    def body(in_ref, out_ref):
        x = in_ref[...]
        for t in range(sper):
            xs = x[t * half:(t + 1) * half]
            out_ref[t] = jnp.concatenate(
                [xs[:, :embed].T, xs[:, embed:].T], axis=1)

    return pl.pallas_call(
        body,
        grid=(seq // sper,),
        in_specs=[pl.BlockSpec((sper * half, 2 * embed), lambda s: (s, 0))],
        out_specs=pl.BlockSpec((sper, embed, batch), lambda s: (s, 0, 0)),
        out_shape=jax.ShapeDtypeStruct((seq, embed, batch), jnp.float32),
    )(rows2)


def kernel(ids, weight):
    batch, seq = ids.shape
    vocab, embed = weight.shape
    total = batch * seq
    assert total % (_NW * _CHUNK * _M) == 0
    n_chunks = total // (_NW * _CHUNK)

    packed = _relayout_table(weight.T)
    table2 = packed.reshape(2 * packed.shape[0], embed)
    ids_perm = ids.T.reshape(seq, 2, batch // 2).transpose(0, 2, 1)
    idp = ids_perm.reshape(total)
    blk = idp // _PBLK
    idp = (blk // 2) * (2 * _PBLK) + (idp % _PBLK) * 2 + (blk % 2)
    ids_flat = idp.reshape(total // _CHUNK, _CHUNK)
    rows = _gather_rows(total, embed, n_chunks)(ids_flat, table2)
    rows2 = rows.reshape(total // 2, 2 * embed)
    o_phys = _format_output(rows2, seq, batch, embed)
    return jnp.transpose(o_phys, (2, 0, 1))


# speedup vs baseline: 2.1955x; 1.0078x over previous
import functools

import jax
import jax.numpy as jnp
from jax import lax
from jax.experimental import pallas as pl
from jax.experimental.pallas import tpu as pltpu
from jax.experimental.pallas import tpu_sc as plsc

_INFO = plsc.get_sparse_core_info()
_NC = _INFO.num_cores
_NS = _INFO.num_subcores
_NW = _NC * _NS

_CHUNK = 128
_M = 10

_PBLK = 2048


def _relayout_table(wt):
    embed, vocab = wt.shape
    n_pairs = pl.cdiv(vocab, 2 * _PBLK)

    def body(in1_ref, in2_ref, out_ref):
        out_ref[...] = jnp.concatenate(
            [in1_ref[...].T, in2_ref[...].T], axis=1)

    return pl.pallas_call(
        body,
        grid=(n_pairs,),
        in_specs=[
            pl.BlockSpec((embed, _PBLK), lambda i: (0, 2 * i)),
            pl.BlockSpec(
                (embed, _PBLK),
                lambda i: (0, jnp.minimum(2 * i + 1,
                                          pl.cdiv(vocab, _PBLK) - 1))),
        ],
        out_specs=pl.BlockSpec((_PBLK, 2 * embed), lambda i: (i, 0)),
        out_shape=jax.ShapeDtypeStruct((n_pairs * _PBLK, 2 * embed),
                                       jnp.float32),
    )(wt, wt)


def _gather_rows(total_rows, embed_dim, n_chunks, row0=0, ring=None):
    _M = ring if ring is not None else 10
    rows_per_w = n_chunks * _CHUNK
    n_groups = n_chunks // _M
    mesh = plsc.VectorSubcoreMesh(core_axis_name="c", subcore_axis_name="s")

    scratch = (
        [pltpu.VMEM((n_chunks, _CHUNK), jnp.int32)]
        + [pltpu.VMEM((_CHUNK, embed_dim), jnp.float32) for _ in range(_M)]
        + [pltpu.SemaphoreType.DMA for _ in range(2 * _M)]
    )

    @functools.partial(
        pl.kernel,
        mesh=mesh,
        out_type=jax.ShapeDtypeStruct((total_rows, embed_dim), jnp.float32),
        scratch_types=scratch,
        compiler_params=pltpu.CompilerParams(use_tc_tiling_on_sc=False),
    )
    def k(ids_hbm, table_hbm, out_hbm, idx_v, *rest):
        bufs = rest[:_M]
        gs = rest[_M:2 * _M]
        ws = rest[2 * _M:]
        wid = lax.axis_index("s") * _NC + lax.axis_index("c")
        base = wid * rows_per_w
        pltpu.sync_copy(
            ids_hbm.at[pl.ds(row0 + wid * n_chunks, n_chunks)], idx_v)

        for b in range(_M):
            pltpu.async_copy(table_hbm.at[idx_v.at[b]], bufs[b], gs[b])

        def group(g, carry):
            for b in range(_M):
                jj = g * _M + b
                pltpu.make_async_copy(table_hbm.at[idx_v.at[b]], bufs[b], gs[b]).wait()
                pltpu.async_copy(
                    bufs[b], out_hbm.at[pl.ds(base + jj * _CHUNK, _CHUNK)], ws[b])
            for b in range(_M):
                jj = (g + 1) * _M + b
                pltpu.make_async_copy(
                    bufs[b], out_hbm.at[pl.ds(base, _CHUNK)], ws[b]).wait()
                pltpu.async_copy(table_hbm.at[idx_v.at[jj]], bufs[b], gs[b])
            return carry

        lax.fori_loop(0, n_groups - 1, group, 0, unroll=False)

        g_last = n_groups - 1
        for b in range(_M):
            jj = g_last * _M + b
            pltpu.make_async_copy(table_hbm.at[idx_v.at[b]], bufs[b], gs[b]).wait()
            pltpu.async_copy(
                bufs[b], out_hbm.at[pl.ds(base + jj * _CHUNK, _CHUNK)], ws[b])
        for b in range(_M):
            pltpu.make_async_copy(
                bufs[b], out_hbm.at[pl.ds(base, _CHUNK)], ws[b]).wait()

    return k


def _format_output(rows2, prev, s_lo, s_cnt, total_seq, batch, embed,
                   sper=5):
    half = batch // 2

    def body(in_ref, *refs):
        out_ref = refs[-1]
        x = in_ref[...]
        for t in range(sper):
            xs = x[t * half:(t + 1) * half]
            out_ref[t] = jnp.concatenate(
                [xs[:, :embed].T, xs[:, embed:].T], axis=1)

    in_specs = [pl.BlockSpec((sper * half, 2 * embed), lambda s: (s, 0))]
    args = (rows2,)
    aliases = {}
    if prev is not None:
        in_specs.append(pl.BlockSpec(memory_space=pltpu.MemorySpace.HBM))
        args = (rows2, prev)
        aliases = {1: 0}

    return pl.pallas_call(
        body,
        grid=(s_cnt // sper,),
        in_specs=in_specs,
        out_specs=pl.BlockSpec(
            (sper, embed, batch), lambda s: (s + s_lo // sper, 0, 0)),
        out_shape=jax.ShapeDtypeStruct((total_seq, embed, batch),
                                       jnp.float32),
        input_output_aliases=aliases,
    )(*args)


def kernel(ids, weight):
    batch, seq = ids.shape
    vocab, embed = weight.shape
    total = batch * seq

    packed = _relayout_table(weight.T)
    table2 = packed.reshape(2 * packed.shape[0], embed)
    ids_perm = ids.T.reshape(seq, 2, batch // 2).transpose(0, 2, 1)
    idp = ids_perm.reshape(total)
    blk = idp // _PBLK
    idp = (blk // 2) * (2 * _PBLK) + (idp % _PBLK) * 2 + (blk % 2)
    ids_flat = idp.reshape(total // _CHUNK, _CHUNK)

    sh = seq // 2
    htot = total // 2
    hchunks = htot // (_NW * _CHUNK)
    rows_a = _gather_rows(htot, embed, hchunks, row0=0, ring=5)(
        ids_flat, table2)
    rows_b = _gather_rows(htot, embed, hchunks, row0=htot // _CHUNK,
                          ring=5)(ids_flat, table2)
    o1 = _format_output(rows_a.reshape(htot // 2, 2 * embed), None,
                        0, sh, seq, batch, embed)
    o2 = _format_output(rows_b.reshape(htot // 2, 2 * embed), o1,
                        sh, sh, seq, batch, embed)
    return jnp.transpose(o2, (2, 0, 1))
